# baseline (speedup 1.0000x reference)
import functools

import jax
import jax.numpy as jnp
from jax import lax
from jax.experimental import pallas as pl
from jax.experimental.pallas import tpu as pltpu
from jax.experimental.pallas import tpu_sc as plsc

H = 1024


def _project_body(lt_ref, tt_ref, w1_ref, w2_ref, b_ref, pl_ref, pt_ref):
    i = pl.program_id(0)
    pt_ref[...] = jnp.dot(tt_ref[...], w2_ref[...],
                          preferred_element_type=jnp.float32)

    @pl.when(i == 0)
    def _():
        pl_ref[...] = jnp.dot(lt_ref[...], w1_ref[...],
                              preferred_element_type=jnp.float32) + b_ref[...]


def _project_tables(layer_table, token_table, w1, w2, b2d):
    n_layers = layer_table.shape[0]
    n_tokens = token_table.shape[0]
    blk = 1024
    grid = n_tokens // blk
    return pl.pallas_call(
        _project_body,
        grid=(grid,),
        in_specs=[
            pl.BlockSpec((n_layers, H), lambda i: (0, 0)),
            pl.BlockSpec((blk, H), lambda i: (i, 0)),
            pl.BlockSpec((H, H), lambda i: (0, 0)),
            pl.BlockSpec((H, H), lambda i: (0, 0)),
            pl.BlockSpec((1, H), lambda i: (0, 0)),
        ],
        out_specs=[
            pl.BlockSpec((n_layers, H), lambda i: (0, 0)),
            pl.BlockSpec((blk, H), lambda i: (i, 0)),
        ],
        out_shape=[
            jax.ShapeDtypeStruct((n_layers, H), jnp.float32),
            jax.ShapeDtypeStruct((n_tokens, H), jnp.float32),
        ],
    )(layer_table, token_table, w1, w2, b2d)


@functools.cache
def _make_gather_add(n_rows, n_layers):
    info = plsc.get_sparse_core_info()
    nc, ns, lanes = info.num_cores, info.num_subcores, info.num_lanes
    nw = nc * ns
    rows_pw = n_rows // nw
    chunk = 32
    n_chunks = rows_pw // chunk
    mesh = plsc.VectorSubcoreMesh(core_axis_name="c", subcore_axis_name="s")

    @functools.partial(
        pl.kernel,
        mesh=mesh,
        compiler_params=pltpu.CompilerParams(
            use_tc_tiling_on_sc=False, needs_layout_passes=False),
        out_type=jax.ShapeDtypeStruct((n_rows, H), jnp.float32),
        scratch_types=[
            pltpu.VMEM((n_layers, H), jnp.float32),
            pltpu.VMEM((rows_pw,), jnp.int32),
            pltpu.VMEM((rows_pw,), jnp.int32),
            pltpu.VMEM((chunk, H), jnp.float32),
            pltpu.SemaphoreType.DMA,
        ],
    )
    def gather_add(li_hbm, ti_hbm, pl_hbm, pt_hbm, out_hbm,
                   ltab, lidx, tidx, rows, sem):
        wid = lax.axis_index("s") * nc + lax.axis_index("c")
        base = wid * rows_pw
        pltpu.sync_copy(pl_hbm, ltab)
        pltpu.sync_copy(li_hbm.at[pl.ds(base, rows_pw)], lidx)
        pltpu.sync_copy(ti_hbm.at[pl.ds(base, rows_pw)], tidx)
        iota = lax.iota(jnp.int32, lanes)

        for g in range(n_chunks):
            off = g * chunk
            pltpu.async_copy(pt_hbm.at[tidx.at[pl.ds(off, chunk)]],
                             rows, sem).wait()
            for r0 in range(0, chunk, lanes):
                li16 = lidx[pl.ds(off + r0, lanes)]
                rowvec = iota + r0

                def col_body(col, carry, li16=li16, rowvec=rowvec):
                    colvec = jnp.full((lanes,), 0, jnp.int32) + col
                    v = plsc.load_gather(ltab, [li16, colvec])
                    plsc.addupdate_scatter(rows, [rowvec, colvec], v)
                    return carry

                lax.fori_loop(0, H, col_body, 0)
            pltpu.sync_copy(rows, out_hbm.at[pl.ds(base + off, chunk)])

    return gather_add


def kernel(layer_indices, token_in_layer_indices, layer_table, token_table,
           W, b):
    if layer_indices.ndim == 1:
        layer_indices = layer_indices[None, :]
        token_in_layer_indices = token_in_layer_indices[None, :]
    bsz, seq = layer_indices.shape
    li = layer_indices.reshape(-1).astype(jnp.int32)
    ti = token_in_layer_indices.reshape(-1).astype(jnp.int32)
    projL, projT = _project_tables(
        layer_table, token_table, W[:H], W[H:], b.reshape(1, H))
    fn = _make_gather_add(li.shape[0], layer_table.shape[0])
    out = fn(li, ti, projL, projT)
    return out.reshape(bsz, seq, H)
